# Initial kernel scaffold; baseline (speedup 1.0000x reference)
import jax
import jax.numpy as jnp
from jax.experimental import pallas as pl


def kernel(x, edge_index, edge_attr, batch, params):
    raise NotImplementedError("write your pallas kernel here")


# speedup vs baseline: 1.0335x; 1.0335x over previous
import functools

import jax
import jax.numpy as jnp
from jax import lax
from jax.experimental import pallas as pl
from jax.experimental.pallas import tpu as pltpu
from jax.experimental.pallas import tpu_sc as plsc

_HI = lax.Precision.HIGHEST

N_NODES = 10000
N_EDGES = 160000
NUM_FEATURES = 128
DIM = 32
NUM_GRAPHS = 128
PAD = 128
NPAD = 10240

_NC = 2
_NS = 16
_NW = _NC * _NS
_CH = 128
_NCHUNK = N_EDGES // _CH
_CPW = -(-_NCHUNK // _NW)
_RPS = NPAD // _NS

_mesh = plsc.VectorSubcoreMesh(
    core_axis_name="c", subcore_axis_name="s", num_cores=_NC, num_subcores=_NS
)

_LOG2 = 0.6931471805599453


def _sc_gather(table, idx):
    @functools.partial(
        pl.kernel,
        out_type=jax.ShapeDtypeStruct((N_EDGES, PAD), jnp.float32),
        mesh=_mesh,
        scratch_types=[
            pltpu.VMEM((_CH,), jnp.int32),
            pltpu.VMEM((_CH, PAD), jnp.float32),
            pltpu.SemaphoreType.DMA,
        ],
    )
    def k(tbl, idxh, outh, idx_v, rows_v, sem):
        wid = lax.axis_index("s") * _NC + lax.axis_index("c")

        @pl.loop(0, _CPW)
        def _(j):
            cid = j * _NW + wid

            @pl.when(cid < _NCHUNK)
            def _():
                base = cid * _CH
                pltpu.sync_copy(idxh.at[pl.ds(base, _CH)], idx_v)
                pltpu.async_copy(tbl.at[idx_v], rows_v, sem).wait()
                pltpu.sync_copy(rows_v, outh.at[pl.ds(base, _CH)])

    return k(table, idx)


def _sc_scatter(msg, dst, zfull):
    @functools.partial(
        pl.kernel,
        out_type=jax.ShapeDtypeStruct((_NC, NPAD, PAD), jnp.float32),
        mesh=_mesh,
        scratch_types=[
            pltpu.VMEM_SHARED((NPAD, PAD), jnp.float32),
            pltpu.VMEM((_CH,), jnp.int32),
            pltpu.VMEM((_CH, PAD), jnp.float32),
        ],
    )
    def k(msgh, dsth, zh, s_out, shared, idx_v, val_v):
        c = lax.axis_index("c")
        s = lax.axis_index("s")
        wid = s * _NC + c
        rs = s * _RPS
        pltpu.sync_copy(zh.at[pl.ds(rs, _RPS)], shared.at[pl.ds(rs, _RPS)])
        plsc.subcore_barrier()

        @pl.loop(0, _CPW)
        def _(j):
            cid = j * _NW + wid

            @pl.when(cid < _NCHUNK)
            def _():
                base = cid * _CH
                pltpu.sync_copy(dsth.at[pl.ds(base, _CH)], idx_v)
                pltpu.sync_copy(msgh.at[pl.ds(base, _CH)], val_v)
                pltpu.sync_copy(val_v, shared.at[idx_v], add=True)

        plsc.subcore_barrier()
        pltpu.sync_copy(shared.at[pl.ds(rs, _RPS)], s_out.at[c, pl.ds(rs, _RPS)])

    return k(msg, dst, zfull)


def _tc_lin0(x, W, b):
    def body(x_ref, w_ref, b_ref, o_ref):
        h = jnp.maximum(
            jnp.dot(x_ref[...], w_ref[...], preferred_element_type=jnp.float32)
            + b_ref[...], 0.0)
        hp = jnp.concatenate(
            [h, jnp.zeros((N_NODES, PAD - DIM), jnp.float32)], axis=1)
        o_ref[...] = jnp.concatenate(
            [hp, jnp.zeros((NPAD - N_NODES, PAD), jnp.float32)], axis=0)

    return pl.pallas_call(
        body, out_shape=jax.ShapeDtypeStruct((NPAD, PAD), jnp.float32),
    )(x, W, b.reshape(1, DIM))


_EB = 1000


def _tc_msg(ea, xsrc, W1, b1, W2, b2):
    def body(ea_ref, xs_ref, w1_ref, b1_ref, w2_ref, b2_ref, o_ref):
        h = jnp.maximum(
            jnp.dot(ea_ref[...], w1_ref[...], preferred_element_type=jnp.float32)
            + b1_ref[...], 0.0)
        we = jnp.dot(h, w2_ref[...], preferred_element_type=jnp.float32) + b2_ref[...]
        xs = xs_ref[...][:, :DIM]
        prod = we.reshape(_EB, DIM, DIM) * xs[:, :, None]
        o_ref[...] = jnp.concatenate(
            [jnp.sum(prod, axis=1),
             jnp.ones((_EB, 1), jnp.float32),
             jnp.zeros((_EB, PAD - DIM - 1), jnp.float32)], axis=1)

    grid = (N_EDGES // _EB,)
    return pl.pallas_call(
        body,
        grid=grid,
        in_specs=[
            pl.BlockSpec((_EB, 5), lambda i: (i, 0)),
            pl.BlockSpec((_EB, PAD), lambda i: (i, 0)),
            pl.BlockSpec((5, 64), lambda i: (0, 0)),
            pl.BlockSpec((1, 64), lambda i: (0, 0)),
            pl.BlockSpec((64, DIM * DIM), lambda i: (0, 0)),
            pl.BlockSpec((1, DIM * DIM), lambda i: (0, 0)),
        ],
        out_specs=pl.BlockSpec((_EB, PAD), lambda i: (i, 0)),
        out_shape=jax.ShapeDtypeStruct((N_EDGES, PAD), jnp.float32),
        compiler_params=pltpu.CompilerParams(dimension_semantics=("parallel",)),
    )(ea, xsrc, W1, b1.reshape(1, 64), W2, b2.reshape(1, DIM * DIM))


_RB = 2048


def _tc_gru(s2, h, conv_b, Wi, bi, Wh, bh):
    def body(s2_ref, h_ref, cb_ref, wi_ref, bi_ref, wh_ref, bh_ref, o_ref):
        sfull = s2_ref[0] + s2_ref[1]
        cnt = sfull[:, DIM:DIM + 1]
        s = sfull[:, :DIM] / jnp.maximum(cnt, 1.0) + cb_ref[...]
        m = jnp.maximum(s, 0.0)
        hh = h_ref[...][:, :DIM]
        gi = jnp.dot(m, wi_ref[...], preferred_element_type=jnp.float32) + bi_ref[...]
        gh = jnp.dot(hh, wh_ref[...], preferred_element_type=jnp.float32) + bh_ref[...]
        r = jax.nn.sigmoid(gi[:, 0:DIM] + gh[:, 0:DIM])
        z = jax.nn.sigmoid(gi[:, DIM:2 * DIM] + gh[:, DIM:2 * DIM])
        n = jnp.tanh(gi[:, 2 * DIM:] + r * gh[:, 2 * DIM:])
        hnew = (1.0 - z) * n + z * hh
        o_ref[...] = jnp.concatenate(
            [hnew, jnp.zeros((_RB, PAD - DIM), jnp.float32)], axis=1)

    return pl.pallas_call(
        body,
        grid=(NPAD // _RB,),
        in_specs=[
            pl.BlockSpec((2, _RB, PAD), lambda i: (0, i, 0)),
            pl.BlockSpec((_RB, PAD), lambda i: (i, 0)),
            pl.BlockSpec((1, DIM), lambda i: (0, 0)),
            pl.BlockSpec((DIM, 3 * DIM), lambda i: (0, 0)),
            pl.BlockSpec((1, 3 * DIM), lambda i: (0, 0)),
            pl.BlockSpec((DIM, 3 * DIM), lambda i: (0, 0)),
            pl.BlockSpec((1, 3 * DIM), lambda i: (0, 0)),
        ],
        out_specs=pl.BlockSpec((_RB, PAD), lambda i: (i, 0)),
        out_shape=jax.ShapeDtypeStruct((NPAD, PAD), jnp.float32),
        compiler_params=pltpu.CompilerParams(dimension_semantics=("parallel",)),
    )(s2, h, conv_b.reshape(1, DIM), Wi, bi.reshape(1, 3 * DIM),
      Wh, bh.reshape(1, 3 * DIM))


def _softplus(t):
    return jnp.maximum(t, 0.0) + jnp.log(1.0 + jnp.exp(-jnp.abs(t)))


def _tc_pool(x, batch_row, lstm_W, lstm_b):
    def body(x_ref, br_ref, lstmw_ref, lstmb_ref, o_ref):
        G = NUM_GRAPHS
        giota = lax.broadcasted_iota(jnp.int32, (G, 1), 0)
        Af = (br_ref[...] == giota).astype(jnp.float32)
        cntg = jnp.sum(Af, axis=1, keepdims=True)
        xm = jnp.dot(Af, x_ref[...], preferred_element_type=jnp.float32,
                     precision=_HI)
        xm = xm / jnp.maximum(cntg, 1.0)
        o_ref[...] = jnp.dot(xm, lstmw_ref[...],
                             preferred_element_type=jnp.float32) + lstmb_ref[...]

    return pl.pallas_call(
        body, out_shape=jax.ShapeDtypeStruct((NUM_GRAPHS, DIM), jnp.float32),
    )(x, batch_row, lstm_W, lstm_b.reshape(1, DIM))


def _tc_tail(out, y_, batch_row, p):
    def body(out_ref, y_ref, br_ref,
             s2swi_ref, s2sbi_ref, s2swh_ref, s2sbh_ref,
             gdw1_ref, gdb1_ref, gdw2_ref, gdb2_ref, gdw3_ref, gdb3_ref,
             gdws_ref, gdbs_ref, gsw_ref, gsb_ref, o_ref):
        G = NUM_GRAPHS
        o = out_ref[...]
        br = br_ref[...]
        giota = lax.broadcasted_iota(jnp.int32, (G, 1), 0)
        giota_r = lax.broadcasted_iota(jnp.int32, (1, G), 1)
        Af = (br == giota).astype(jnp.float32)

        h_ = jnp.zeros((G, DIM), jnp.float32)
        c_ = jnp.zeros((G, DIM), jnp.float32)
        q = jnp.zeros((G, 2 * DIM), jnp.float32)
        for _ in range(3):
            g = (jnp.dot(q, s2swi_ref[...], preferred_element_type=jnp.float32)
                 + s2sbi_ref[...]
                 + jnp.dot(h_, s2swh_ref[...], preferred_element_type=jnp.float32)
                 + s2sbh_ref[...])
            ig = jax.nn.sigmoid(g[:, 0:DIM])
            fg = jax.nn.sigmoid(g[:, DIM:2 * DIM])
            gg = jnp.tanh(g[:, 2 * DIM:3 * DIM])
            og = jax.nn.sigmoid(g[:, 3 * DIM:])
            c_ = fg * c_ + ig * gg
            h_ = og * jnp.tanh(c_)

            hbT = jnp.dot(jnp.transpose(h_), Af,
                          preferred_element_type=jnp.float32,
                          precision=_HI)
            eT = jnp.sum(jnp.transpose(o) * hbT, axis=0,
                         keepdims=True)
            masked = eT * Af + (Af - 1.0) * 1e30
            m = jnp.max(masked, axis=1, keepdims=True)
            EX = jnp.exp(masked - m) * Af
            den = jnp.sum(EX, axis=1, keepdims=True)
            A = EX / (den + 1e-16)
            r = jnp.dot(A, o, preferred_element_type=jnp.float32,
                        precision=_HI)
            q = jnp.concatenate([h_, r], axis=1)

        y_ = y_ref[...]

        b1 = jnp.maximum(jnp.dot(q, gdw1_ref[...],
                                 preferred_element_type=jnp.float32)
                         + gdb1_ref[...], 0.0)
        b2 = jnp.maximum(jnp.dot(b1, gdw2_ref[...],
                                 preferred_element_type=jnp.float32)
                         + gdb2_ref[...], 0.0)
        b3 = jnp.maximum(jnp.dot(b2, gdw3_ref[...],
                                 preferred_element_type=jnp.float32)
                         + gdb3_ref[...], 0.0)
        g_enc = b3 + jnp.dot(q, gdws_ref[...],
                             preferred_element_type=jnp.float32) + gdbs_ref[...]
        s_enc = jnp.dot(y_, gsw_ref[...], preferred_element_type=jnp.float32) \
            + gsb_ref[...]
        res = lax.dot_general(g_enc, s_enc, (((1,), (1,)), ((), ())),
                              preferred_element_type=jnp.float32)

        eye = (giota == giota_r).astype(jnp.float32)
        pos = res * eye
        neg = res * (1.0 - eye)
        E_pos = jnp.sum(_LOG2 - _softplus(-pos)) / G
        E_neg = jnp.sum(_softplus(-neg) + neg - _LOG2) / (G * (G - 1))
        o_ref[...] = (E_neg - E_pos).reshape(1, 1)

    return pl.pallas_call(
        body, out_shape=jax.ShapeDtypeStruct((1, 1), jnp.float32),
    )(out, y_, batch_row,
      p['s2s_Wi'], p['s2s_bi'].reshape(1, 4 * DIM),
      p['s2s_Wh'], p['s2s_bh'].reshape(1, 4 * DIM),
      p['gd_W1'], p['gd_b1'].reshape(1, DIM),
      p['gd_W2'], p['gd_b2'].reshape(1, DIM),
      p['gd_W3'], p['gd_b3'].reshape(1, DIM),
      p['gd_Ws'], p['gd_bs'].reshape(1, DIM),
      p['gs_W'], p['gs_b'].reshape(1, DIM))


def kernel(x, edge_index, edge_attr, batch, params):
    p = params
    src = edge_index[0]
    dst = edge_index[1]
    zfull = jnp.zeros((NPAD, PAD), jnp.float32)

    out = _tc_lin0(x, p['lin0_W'], p['lin0_b'])
    h = out
    for it in range(3):
        xsrc = _sc_gather(out, src)
        msg = _tc_msg(edge_attr, xsrc, p['enn_W1'], p['enn_b1'],
                      p['enn_W2'], p['enn_b2'])
        s2 = _sc_scatter(msg, dst, zfull)
        h = _tc_gru(s2, h, p['conv_b'], p['gru_Wi'], p['gru_bi'],
                    p['gru_Wh'], p['gru_bh'])
        out = h
    br = batch.reshape(1, N_NODES)
    y_ = _tc_pool(x, br, p['lstm_W'], p['lstm_b'])
    out32 = lax.slice(out, (0, 0), (N_NODES, DIM))
    loss = _tc_tail(out32, y_, br, p)
    return loss.reshape(())
